# Initial kernel scaffold; baseline (speedup 1.0000x reference)
import jax
import jax.numpy as jnp
from jax.experimental import pallas as pl


def kernel(x, edge_index, edge_weight, Wl, bl, Wr, br, att, bias, W2, b2):
    raise NotImplementedError("write your pallas kernel here")


# speedup vs baseline: 7.9353x; 7.9353x over previous
import functools

import jax
import jax.numpy as jnp
from jax import lax
from jax.experimental import pallas as pl
from jax.experimental.pallas import tpu as pltpu
from jax.experimental.pallas import tpu_sc as plsc

D = 128
CW = 144
NEG = 0.2
NC, NS = 2, 16
NW = NC * NS
K = 128
BLK = 256


def _prep_body(x_ref, wl_ref, wr_ref, bl_ref, br_ref, att_ref, xl_ref, xr_ref):
    xb = x_ref[...]
    xl = jnp.dot(xb, wl_ref[...], preferred_element_type=jnp.float32) + bl_ref[...]
    xr = jnp.dot(xb, wr_ref[...], preferred_element_type=jnp.float32) + br_ref[...]
    z = xl + xr
    z = jnp.maximum(z, NEG * z)
    m = jnp.sum(z * att_ref[...], axis=1, keepdims=True)
    b = xb.shape[0]
    sh = m * (1.0 / 1.2)
    xl_ref[...] = jnp.concatenate(
        [xl, jnp.ones((b, 1), jnp.float32), jnp.zeros((b, CW - D - 1), jnp.float32)],
        axis=1)
    xr_ref[...] = jnp.concatenate(
        [xr, sh - 1.0, -sh, jnp.zeros((b, CW - D - 2), jnp.float32)], axis=1)


def _tc_prep(xp, Wl, bl, Wr, br, att):
    np_ = xp.shape[0]
    grid = (np_ // BLK,)
    return pl.pallas_call(
        _prep_body,
        grid=grid,
        in_specs=[
            pl.BlockSpec((BLK, D), lambda i: (i, 0)),
            pl.BlockSpec((D, D), lambda i: (0, 0)),
            pl.BlockSpec((D, D), lambda i: (0, 0)),
            pl.BlockSpec((1, D), lambda i: (0, 0)),
            pl.BlockSpec((1, D), lambda i: (0, 0)),
            pl.BlockSpec((1, D), lambda i: (0, 0)),
        ],
        out_specs=[
            pl.BlockSpec((BLK, CW), lambda i: (i, 0)),
            pl.BlockSpec((BLK, CW), lambda i: (i, 0)),
        ],
        out_shape=[
            jax.ShapeDtypeStruct((np_, CW), jnp.float32),
            jax.ShapeDtypeStruct((np_, CW), jnp.float32),
        ],
    )(xp, Wl, Wr, bl.reshape(1, D), br.reshape(1, D), att.reshape(1, D))


def _sc_edge_fn(np_, ep):
    chunks = ep // (NW * K)
    rows_per_tile = np_ // NS
    mesh = plsc.VectorSubcoreMesh(core_axis_name="c", subcore_axis_name="s",
                                  num_cores=NC, num_subcores=NS)

    @functools.partial(
        pl.kernel,
        out_type=jax.ShapeDtypeStruct((NC, np_, CW), jnp.float32),
        mesh=mesh,
        compiler_params=pltpu.CompilerParams(use_tc_tiling_on_sc=False),
        scratch_types=[
            pltpu.VMEM((K,), jnp.int32),
            pltpu.VMEM((1, K), jnp.int32),
            pltpu.VMEM((K, CW), jnp.float32),
            pltpu.VMEM((K, CW), jnp.float32),
            pltpu.VMEM((CW,), jnp.float32),
            pltpu.VMEM_SHARED((np_, CW), jnp.float32),
            pltpu.SemaphoreType.DMA,
            pltpu.SemaphoreType.DMA,
        ],
    )
    def sc_edge(xl_hbm, xr_hbm, src_hbm, dst_hbm, zero_hbm, att_hbm, out_hbm,
                srcv, dstv, ubuf, vbuf, attv, accsh, sem1, sem2):
        cid = lax.axis_index("c")
        sid = lax.axis_index("s")
        wid = sid * NC + cid
        pltpu.sync_copy(zero_hbm, accsh.at[pl.ds(sid * rows_per_tile, rows_per_tile)])
        pltpu.sync_copy(att_hbm, attv)
        att_regs = [attv[pl.ds(16 * j, 16)] for j in range(CW // 16)]
        plsc.subcore_barrier()

        def chunk_body(ci, carry):
            base = wid * (chunks * K) + ci * K
            pltpu.sync_copy(src_hbm.at[pl.ds(base, K)], srcv)
            pltpu.sync_copy(dst_hbm.at[pl.ds(base, K)], dstv.at[0])
            cp1 = pltpu.async_copy(xl_hbm.at[srcv], ubuf, sem1)
            cp2 = pltpu.async_copy(xr_hbm.at[dstv.at[0]], vbuf, sem2)
            cp1.wait()
            cp2.wait()

            def edge_body(e, c2):
                us = [ubuf[e, pl.ds(16 * j, 16)] for j in range(CW // 16)]
                s = jnp.zeros((16,), jnp.float32)
                for j in range(CW // 16):
                    z = us[j] + vbuf[e, pl.ds(16 * j, 16)]
                    z = jnp.maximum(z, NEG * z)
                    s = s + z * att_regs[j]
                s = s + lax.rev(s, (0,))
                t = s[0]
                for i in range(1, 8):
                    t = t + s[i]
                wv = jnp.exp(jnp.full((16,), t, jnp.float32))
                for j in range(CW // 16):
                    ubuf[e, pl.ds(16 * j, 16)] = us[j] * wv
                return c2

            lax.fori_loop(0, K, edge_body, 0)
            pltpu.sync_copy(ubuf, accsh.at[dstv.at[0]], add=True)
            return carry

        lax.fori_loop(0, chunks, chunk_body, 0)
        plsc.subcore_barrier()
        pltpu.sync_copy(
            accsh.at[pl.ds(sid * rows_per_tile, rows_per_tile)],
            out_hbm.at[cid, pl.ds(sid * rows_per_tile, rows_per_tile)])

    return sc_edge


def _final_body(acc_ref, xl_ref, bias_ref, w2_ref, b2_ref, o_ref):
    a = acc_ref[0] + acc_ref[1]
    num = a[:, 0:D] + xl_ref[:, 0:D]
    den = a[:, D:D + 1] + (1.0 + 1e-16)
    h = num / den + bias_ref[...]
    h = jnp.where(h > 0, h, jnp.exp(jnp.minimum(h, 0.0)) - 1.0)
    o_ref[...] = jnp.dot(h, w2_ref[...], preferred_element_type=jnp.float32) \
        + b2_ref[...]


def _tc_final(accs, XL, bias, W2, b2):
    np_ = XL.shape[0]
    grid = (np_ // BLK,)
    return pl.pallas_call(
        _final_body,
        grid=grid,
        in_specs=[
            pl.BlockSpec((NC, BLK, CW), lambda i: (0, i, 0)),
            pl.BlockSpec((BLK, CW), lambda i: (i, 0)),
            pl.BlockSpec((1, D), lambda i: (0, 0)),
            pl.BlockSpec((D, D), lambda i: (0, 0)),
            pl.BlockSpec((1, D), lambda i: (0, 0)),
        ],
        out_specs=pl.BlockSpec((BLK, D), lambda i: (i, 0)),
        out_shape=jax.ShapeDtypeStruct((np_, D), jnp.float32),
    )(accs, XL, bias.reshape(1, D), W2, b2.reshape(1, D))


def kernel(x, edge_index, edge_weight, Wl, bl, Wr, br, att, bias, W2, b2):
    n = x.shape[0]
    e = edge_index.shape[1]
    np_ = ((n + 16) + 255) // 256 * 256
    ep = (e + (NW * K) - 1) // (NW * K) * (NW * K)

    xp = jnp.pad(x, ((0, np_ - n), (0, 0)))
    dummy = jnp.full((ep - e,), n, jnp.int32)
    srcp = jnp.concatenate([edge_index[0], dummy])
    dstp = jnp.concatenate([edge_index[1], dummy])
    att_ext = jnp.concatenate(
        [att, jnp.array([-1.0, 1.0], jnp.float32),
         jnp.zeros((CW - D - 2,), jnp.float32)])

    XL, XR = _tc_prep(xp, Wl, bl, Wr, br, att)
    zeros_tile = jnp.zeros((np_ // NS, CW), jnp.float32)
    accs = _sc_edge_fn(np_, ep)(XL, XR, srcp, dstp, zeros_tile, att_ext)
    outp = _tc_final(accs, XL, bias, W2, b2)
    return outp[:n]


# speedup vs baseline: 13.4037x; 1.6891x over previous
import functools

import jax
import jax.numpy as jnp
from jax import lax
from jax.experimental import pallas as pl
from jax.experimental.pallas import tpu as pltpu
from jax.experimental.pallas import tpu_sc as plsc

D = 128
CW = 144
NEG = 0.2
NC, NS = 2, 16
NW = NC * NS
K = 64
BLK = 256


def _prep_body(x_ref, wl_ref, wr_ref, bl_ref, br_ref, att_ref, xl_ref, xr_ref):
    xb = x_ref[...]
    xl = jnp.dot(xb, wl_ref[...], preferred_element_type=jnp.float32) + bl_ref[...]
    xr = jnp.dot(xb, wr_ref[...], preferred_element_type=jnp.float32) + br_ref[...]
    z = xl + xr
    z = jnp.maximum(z, NEG * z)
    m = jnp.sum(z * att_ref[...], axis=1, keepdims=True)
    b = xb.shape[0]
    sh = m * (1.0 / 1.2)
    xl_ref[...] = jnp.concatenate(
        [xl, jnp.ones((b, 1), jnp.float32), jnp.zeros((b, CW - D - 1), jnp.float32)],
        axis=1)
    xr_ref[...] = jnp.concatenate(
        [xr, sh - 1.0, -sh, jnp.zeros((b, CW - D - 2), jnp.float32)], axis=1)


def _tc_prep(xp, Wl, bl, Wr, br, att):
    np_ = xp.shape[0]
    grid = (np_ // BLK,)
    return pl.pallas_call(
        _prep_body,
        grid=grid,
        in_specs=[
            pl.BlockSpec((BLK, D), lambda i: (i, 0)),
            pl.BlockSpec((D, D), lambda i: (0, 0)),
            pl.BlockSpec((D, D), lambda i: (0, 0)),
            pl.BlockSpec((1, D), lambda i: (0, 0)),
            pl.BlockSpec((1, D), lambda i: (0, 0)),
            pl.BlockSpec((1, D), lambda i: (0, 0)),
        ],
        out_specs=[
            pl.BlockSpec((BLK, CW), lambda i: (i, 0)),
            pl.BlockSpec((BLK, CW), lambda i: (i, 0)),
        ],
        out_shape=[
            jax.ShapeDtypeStruct((np_, CW), jnp.float32),
            jax.ShapeDtypeStruct((np_, CW), jnp.float32),
        ],
    )(xp, Wl, Wr, bl.reshape(1, D), br.reshape(1, D), att.reshape(1, D))


def _sc_edge_fn(np_, ep):
    chunks = ep // (NW * K)
    assert chunks % 2 == 0
    rows_per_tile = np_ // NS
    mesh = plsc.VectorSubcoreMesh(core_axis_name="c", subcore_axis_name="s",
                                  num_cores=NC, num_subcores=NS)

    @functools.partial(
        pl.kernel,
        out_type=jax.ShapeDtypeStruct((NC, np_, CW), jnp.float32),
        mesh=mesh,
        compiler_params=pltpu.CompilerParams(use_tc_tiling_on_sc=False),
        scratch_types=[
            pltpu.VMEM((2, K), jnp.int32),
            pltpu.VMEM((2, K), jnp.int32),
            pltpu.VMEM((K, CW), jnp.float32),
            pltpu.VMEM((K, CW), jnp.float32),
            pltpu.VMEM((K, CW), jnp.float32),
            pltpu.VMEM((K, CW), jnp.float32),
            pltpu.VMEM((CW,), jnp.float32),
            pltpu.VMEM_SHARED((np_, CW), jnp.float32),
            pltpu.SemaphoreType.DMA,
            pltpu.SemaphoreType.DMA,
            pltpu.SemaphoreType.DMA,
            pltpu.SemaphoreType.DMA,
            pltpu.SemaphoreType.DMA,
            pltpu.SemaphoreType.DMA,
        ],
    )
    def sc_edge(xl_hbm, xr_hbm, src_hbm, dst_hbm, zero_hbm, att_hbm, out_hbm,
                srcv, dstv, ub0, ub1, vb0, vb1, attv, accsh,
                sgu0, sgu1, sgv0, sgv1, ss0, ss1):
        cid = lax.axis_index("c")
        sid = lax.axis_index("s")
        wid = sid * NC + cid
        ub = [ub0, ub1]
        vb = [vb0, vb1]
        sem_gu = [sgu0, sgu1]
        sem_gv = [sgv0, sgv1]
        sem_s = [ss0, ss1]
        pltpu.sync_copy(zero_hbm, accsh.at[pl.ds(sid * rows_per_tile, rows_per_tile)])
        pltpu.sync_copy(att_hbm, attv)
        att_regs = [attv[pl.ds(16 * j, 16)] for j in range(CW // 16)]
        iof = lax.iota(jnp.int32, 16).astype(jnp.float32)
        u8c = jnp.maximum(1.0 - iof, 0.0)
        plsc.subcore_barrier()

        def load_idx(p, c):
            base = wid * (chunks * K) + c * K
            pltpu.sync_copy(src_hbm.at[pl.ds(base, K)], srcv.at[p])
            pltpu.sync_copy(dst_hbm.at[pl.ds(base, K)], dstv.at[p])

        def gather_start(p):
            pltpu.async_copy(xl_hbm.at[srcv.at[p]], ub[p], sem_gu[p])
            pltpu.async_copy(xr_hbm.at[dstv.at[p]], vb[p], sem_gv[p])

        def gather_wait(p):
            pltpu.make_async_copy(xl_hbm.at[srcv.at[p]], ub[p], sem_gu[p]).wait()
            pltpu.make_async_copy(xr_hbm.at[dstv.at[p]], vb[p], sem_gv[p]).wait()

        def scatter_start(p):
            pltpu.async_copy(ub[p], accsh.at[dstv.at[p]], sem_s[p], add=True)

        def scatter_wait(p):
            pltpu.make_async_copy(ub[p], accsh.at[dstv.at[p]], sem_s[p]).wait()

        def compute(p):
            up, vp = ub[p], vb[p]

            @plsc.parallel_loop(0, K, step=1, unroll=2)
            def _edge(e):
                us = [up[e, pl.ds(16 * j, 16)] for j in range(8)]
                acc = [None, None, None]
                for j in range(8):
                    z = us[j] + vp[e, pl.ds(16 * j, 16)]
                    z = jnp.maximum(z, NEG * z)
                    za = z * att_regs[j]
                    k = j % 3
                    acc[k] = za if acc[k] is None else acc[k] + za
                z8 = u8c + vp[e, pl.ds(128, 16)]
                z8 = jnp.maximum(z8, NEG * z8)
                s = (acc[0] + acc[1]) + (acc[2] + z8 * att_regs[8])
                s = s + lax.rev(s, (0,))
                t = ((s[0] + s[1]) + (s[2] + s[3])) \
                    + ((s[4] + s[5]) + (s[6] + s[7]))
                wv = jnp.exp(jnp.full((16,), t, jnp.float32))
                for j in range(8):
                    up[e, pl.ds(16 * j, 16)] = us[j] * wv
                up[e, pl.ds(128, 16)] = u8c * wv

        nhalf = chunks // 2
        load_idx(0, 0)
        gather_start(0)

        def outer(cc, carry):
            c0 = 2 * cc
            @pl.when(cc > 0)
            def _():
                scatter_wait(1)

            load_idx(1, c0 + 1)
            gather_start(1)
            gather_wait(0)
            compute(0)
            scatter_start(0)

            @pl.when(cc < nhalf - 1)
            def _():
                scatter_wait(0)
                load_idx(0, c0 + 2)
                gather_start(0)

            gather_wait(1)
            compute(1)
            scatter_start(1)
            return carry

        lax.fori_loop(0, nhalf, outer, 0)
        scatter_wait(0)
        scatter_wait(1)
        plsc.subcore_barrier()
        pltpu.sync_copy(
            accsh.at[pl.ds(sid * rows_per_tile, rows_per_tile)],
            out_hbm.at[cid, pl.ds(sid * rows_per_tile, rows_per_tile)])

    return sc_edge


def _final_body(acc_ref, xl_ref, bias_ref, w2_ref, b2_ref, o_ref):
    a = acc_ref[0] + acc_ref[1]
    num = a[:, 0:D] + xl_ref[:, 0:D]
    den = a[:, D:D + 1] + (1.0 + 1e-16)
    h = num / den + bias_ref[...]
    h = jnp.where(h > 0, h, jnp.exp(jnp.minimum(h, 0.0)) - 1.0)
    o_ref[...] = jnp.dot(h, w2_ref[...], preferred_element_type=jnp.float32) \
        + b2_ref[...]


def _tc_final(accs, XL, bias, W2, b2):
    np_ = XL.shape[0]
    grid = (np_ // BLK,)
    return pl.pallas_call(
        _final_body,
        grid=grid,
        in_specs=[
            pl.BlockSpec((NC, BLK, CW), lambda i: (0, i, 0)),
            pl.BlockSpec((BLK, CW), lambda i: (i, 0)),
            pl.BlockSpec((1, D), lambda i: (0, 0)),
            pl.BlockSpec((D, D), lambda i: (0, 0)),
            pl.BlockSpec((1, D), lambda i: (0, 0)),
        ],
        out_specs=pl.BlockSpec((BLK, D), lambda i: (i, 0)),
        out_shape=jax.ShapeDtypeStruct((np_, D), jnp.float32),
    )(accs, XL, bias.reshape(1, D), W2, b2.reshape(1, D))


def kernel(x, edge_index, edge_weight, Wl, bl, Wr, br, att, bias, W2, b2):
    n = x.shape[0]
    e = edge_index.shape[1]
    np_ = ((n + 16) + 255) // 256 * 256
    ep = (e + (2 * NW * K) - 1) // (2 * NW * K) * (2 * NW * K)

    xp = jnp.pad(x, ((0, np_ - n), (0, 0)))
    dummy = jnp.full((ep - e,), n, jnp.int32)
    srcp = jnp.concatenate([edge_index[0], dummy])
    dstp = jnp.concatenate([edge_index[1], dummy])
    att_ext = jnp.concatenate(
        [att, jnp.array([-1.0, 1.0], jnp.float32),
         jnp.zeros((CW - D - 2,), jnp.float32)])

    XL, XR = _tc_prep(xp, Wl, bl, Wr, br, att)
    zeros_tile = jnp.zeros((np_ // NS, CW), jnp.float32)
    accs = _sc_edge_fn(np_, ep)(XL, XR, srcp, dstp, zeros_tile, att_ext)
    outp = _tc_final(accs, XL, bias, W2, b2)
    return outp[:n]


# speedup vs baseline: 17.2596x; 1.2877x over previous
import functools

import jax
import jax.numpy as jnp
from jax import lax
from jax.experimental import pallas as pl
from jax.experimental.pallas import tpu as pltpu
from jax.experimental.pallas import tpu_sc as plsc

D = 128
CW = 144
NEG = 0.2
NC, NS = 2, 16
NW = NC * NS
K = 48
SUP = 30
NUB = 3
NVB = 2
STEP = 6
BLK = 256


def _prep_body(x_ref, wl_ref, wr_ref, bl_ref, br_ref, att_ref, xl_ref, xr_ref):
    xb = x_ref[...]
    xl = jnp.dot(xb, wl_ref[...], preferred_element_type=jnp.float32) + bl_ref[...]
    xr = jnp.dot(xb, wr_ref[...], preferred_element_type=jnp.float32) + br_ref[...]
    z = xl + xr
    z = jnp.maximum(z, NEG * z)
    m = jnp.sum(z * att_ref[...], axis=1, keepdims=True)
    b = xb.shape[0]
    sh = m * (1.0 / 1.2)
    xl_ref[...] = jnp.concatenate(
        [xl, jnp.ones((b, 1), jnp.float32), jnp.zeros((b, CW - D - 1), jnp.float32)],
        axis=1)
    xr_ref[...] = jnp.concatenate(
        [xr, sh - 1.0, -sh, jnp.zeros((b, CW - D - 2), jnp.float32)], axis=1)


def _tc_prep(xp, Wl, bl, Wr, br, att):
    np_ = xp.shape[0]
    grid = (np_ // BLK,)
    return pl.pallas_call(
        _prep_body,
        grid=grid,
        in_specs=[
            pl.BlockSpec((BLK, D), lambda i: (i, 0)),
            pl.BlockSpec((D, D), lambda i: (0, 0)),
            pl.BlockSpec((D, D), lambda i: (0, 0)),
            pl.BlockSpec((1, D), lambda i: (0, 0)),
            pl.BlockSpec((1, D), lambda i: (0, 0)),
            pl.BlockSpec((1, D), lambda i: (0, 0)),
        ],
        out_specs=[
            pl.BlockSpec((BLK, CW), lambda i: (i, 0)),
            pl.BlockSpec((BLK, CW), lambda i: (i, 0)),
        ],
        out_shape=[
            jax.ShapeDtypeStruct((np_, CW), jnp.float32),
            jax.ShapeDtypeStruct((np_, CW), jnp.float32),
        ],
    )(xp, Wl, Wr, bl.reshape(1, D), br.reshape(1, D), att.reshape(1, D))


def _sc_edge_fn(np_, ep):
    chunks = ep // (NW * K)
    assert chunks % SUP == 0 and SUP % STEP == 0
    nsuper = chunks // SUP
    nii = SUP // STEP
    rows_per_tile = np_ // NS
    mesh = plsc.VectorSubcoreMesh(core_axis_name="c", subcore_axis_name="s",
                                  num_cores=NC, num_subcores=NS)

    @functools.partial(
        pl.kernel,
        out_type=jax.ShapeDtypeStruct((NC, np_, CW), jnp.float32),
        mesh=mesh,
        compiler_params=pltpu.CompilerParams(use_tc_tiling_on_sc=False),
        scratch_types=[
            pltpu.VMEM((SUP, K), jnp.int32),
            pltpu.VMEM((SUP, K), jnp.int32),
            pltpu.VMEM((K, CW), jnp.float32),
            pltpu.VMEM((K, CW), jnp.float32),
            pltpu.VMEM((K, CW), jnp.float32),
            pltpu.VMEM((K, CW), jnp.float32),
            pltpu.VMEM((K, CW), jnp.float32),
            pltpu.VMEM((CW,), jnp.float32),
            pltpu.VMEM_SHARED((np_, CW), jnp.float32),
            pltpu.SemaphoreType.DMA,
            pltpu.SemaphoreType.DMA,
            pltpu.SemaphoreType.DMA,
            pltpu.SemaphoreType.DMA,
            pltpu.SemaphoreType.DMA,
            pltpu.SemaphoreType.DMA,
            pltpu.SemaphoreType.DMA,
            pltpu.SemaphoreType.DMA,
        ],
    )
    def sc_edge(xl_hbm, xr_hbm, src_hbm, dst_hbm, zero_hbm, att_hbm, out_hbm,
                srcv, dstv, ub0, ub1, ub2, vb0, vb1, attv, accsh,
                sgu0, sgu1, sgu2, sgv0, sgv1, ss0, ss1, ss2):
        cid = lax.axis_index("c")
        sid = lax.axis_index("s")
        wid = sid * NC + cid
        ub = [ub0, ub1, ub2]
        vb = [vb0, vb1]
        sem_gu = [sgu0, sgu1, sgu2]
        sem_gv = [sgv0, sgv1]
        sem_s = [ss0, ss1, ss2]
        pltpu.sync_copy(zero_hbm, accsh.at[pl.ds(sid * rows_per_tile, rows_per_tile)])
        pltpu.sync_copy(att_hbm, attv)
        att_regs = [attv[pl.ds(16 * j, 16)] for j in range(CW // 16)]
        iof = lax.iota(jnp.int32, 16).astype(jnp.float32)
        u8c = jnp.maximum(1.0 - iof, 0.0)
        plsc.subcore_barrier()

        def gather_start(uslot, vslot, row):
            pltpu.async_copy(xl_hbm.at[srcv.at[row]], ub[uslot], sem_gu[uslot])
            pltpu.async_copy(xr_hbm.at[dstv.at[row]], vb[vslot], sem_gv[vslot])

        def gather_wait(uslot, vslot, row):
            pltpu.make_async_copy(xl_hbm.at[srcv.at[row]], ub[uslot],
                                  sem_gu[uslot]).wait()
            pltpu.make_async_copy(xr_hbm.at[dstv.at[row]], vb[vslot],
                                  sem_gv[vslot]).wait()

        def scatter_start(uslot, row):
            pltpu.async_copy(ub[uslot], accsh.at[dstv.at[row]], sem_s[uslot],
                             add=True)

        def scatter_wait(uslot, row):
            pltpu.make_async_copy(ub[uslot], accsh.at[dstv.at[row]],
                                  sem_s[uslot]).wait()

        def compute(uslot, vslot):
            up, vp = ub[uslot], vb[vslot]

            @plsc.parallel_loop(0, K, step=1, unroll=1)
            def _edge(e):
                us = [up[e, pl.ds(16 * j, 16)] for j in range(8)]
                acc = [None, None, None]
                for j in range(8):
                    z = us[j] + vp[e, pl.ds(16 * j, 16)]
                    z = jnp.maximum(z, NEG * z)
                    za = z * att_regs[j]
                    k = j % 3
                    acc[k] = za if acc[k] is None else acc[k] + za
                z8 = u8c + vp[e, pl.ds(128, 16)]
                z8 = jnp.maximum(z8, NEG * z8)
                s = (acc[0] + acc[1]) + (acc[2] + z8 * att_regs[8])
                s = s + lax.rev(s, (0,))
                t = ((s[0] + s[1]) + (s[2] + s[3])) \
                    + ((s[4] + s[5]) + (s[6] + s[7]))
                wv = jnp.exp(jnp.full((16,), t, jnp.float32))
                for j in range(8):
                    up[e, pl.ds(16 * j, 16)] = us[j] * wv
                up[e, pl.ds(128, 16)] = u8c * wv

        def super_body(sp, carry):
            idx_base = wid * chunks + sp * SUP
            pltpu.sync_copy(src_hbm.at[pl.ds(idx_base, SUP)], srcv)
            pltpu.sync_copy(dst_hbm.at[pl.ds(idx_base, SUP)], dstv)
            gather_start(0, 0, 0)
            gather_start(1, 1, 1)

            def inner(ii, c2):
                cc0 = ii * STEP
                for ph in range(STEP):
                    cc = cc0 + ph
                    uslot, vslot = ph % NUB, ph % NVB
                    gather_wait(uslot, vslot, cc)
                    compute(uslot, vslot)
                    scatter_start(uslot, cc)
                    nslot = (ph + 2) % NUB
                    if ph == 0:
                        @pl.when(ii > 0)
                        def _():
                            scatter_wait(nslot, cc0 - 1)
                    else:
                        scatter_wait(nslot, cc - 1)
                    if ph >= STEP - 2:
                        @pl.when(ii < nii - 1)
                        def _():
                            gather_start(nslot, ph % NVB, cc + 2)
                    else:
                        gather_start(nslot, ph % NVB, cc + 2)
                return c2

            lax.fori_loop(0, nii, inner, 0)
            scatter_wait((SUP - 1) % NUB, SUP - 1)
            return carry

        lax.fori_loop(0, nsuper, super_body, 0)
        plsc.subcore_barrier()
        pltpu.sync_copy(
            accsh.at[pl.ds(sid * rows_per_tile, rows_per_tile)],
            out_hbm.at[cid, pl.ds(sid * rows_per_tile, rows_per_tile)])

    return sc_edge


def _final_body(acc_ref, xl_ref, bias_ref, w2_ref, b2_ref, o_ref):
    a = acc_ref[0] + acc_ref[1]
    num = a[:, 0:D] + xl_ref[:, 0:D]
    den = a[:, D:D + 1] + (1.0 + 1e-16)
    h = num / den + bias_ref[...]
    h = jnp.where(h > 0, h, jnp.exp(jnp.minimum(h, 0.0)) - 1.0)
    o_ref[...] = jnp.dot(h, w2_ref[...], preferred_element_type=jnp.float32) \
        + b2_ref[...]


def _tc_final(accs, XL, bias, W2, b2):
    np_ = XL.shape[0]
    grid = (np_ // BLK,)
    return pl.pallas_call(
        _final_body,
        grid=grid,
        in_specs=[
            pl.BlockSpec((NC, BLK, CW), lambda i: (0, i, 0)),
            pl.BlockSpec((BLK, CW), lambda i: (i, 0)),
            pl.BlockSpec((1, D), lambda i: (0, 0)),
            pl.BlockSpec((D, D), lambda i: (0, 0)),
            pl.BlockSpec((1, D), lambda i: (0, 0)),
        ],
        out_specs=pl.BlockSpec((BLK, D), lambda i: (i, 0)),
        out_shape=jax.ShapeDtypeStruct((np_, D), jnp.float32),
    )(accs, XL, bias.reshape(1, D), W2, b2.reshape(1, D))


def kernel(x, edge_index, edge_weight, Wl, bl, Wr, br, att, bias, W2, b2):
    n = x.shape[0]
    e = edge_index.shape[1]
    np_ = ((n + 16) + 255) // 256 * 256
    gran = NW * K * SUP
    ep = (e + gran - 1) // gran * gran

    xp = jnp.pad(x, ((0, np_ - n), (0, 0)))
    dummy = jnp.full((ep - e,), n, jnp.int32)
    srcp = jnp.concatenate([edge_index[0], dummy]).reshape(ep // K, K)
    dstp = jnp.concatenate([edge_index[1], dummy]).reshape(ep // K, K)
    att_ext = jnp.concatenate(
        [att, jnp.array([-1.0, 1.0], jnp.float32),
         jnp.zeros((CW - D - 2,), jnp.float32)])

    XL, XR = _tc_prep(xp, Wl, bl, Wr, br, att)
    zeros_tile = jnp.zeros((np_ // NS, CW), jnp.float32)
    accs = _sc_edge_fn(np_, ep)(XL, XR, srcp, dstp, zeros_tile, att_ext)
    outp = _tc_final(accs, XL, bias, W2, b2)
    return outp[:n]
